# baseline (speedup 1.0000x reference)
import jax
import jax.numpy as jnp
from jax import lax
from jax.experimental import pallas as pl
from jax.experimental.pallas import tpu as pltpu
from jax.experimental.pallas import tpu_sc as plsc

N, E, H, G = 10000, 320000, 128, 64
NC, NS, LANES = 2, 16, 16
NW = NC * NS
C = 120
TOT = 84
EPT = TOT * C
EPAD = NW * EPT
TCH = NW * TOT
NACC = 10240
RPT = NACC // NS
WBC = 80
NWB = RPT // WBC
KH = H // LANES
BLK = 2000
NBLK = N // BLK
F32 = jnp.float32

_mesh = plsc.VectorSubcoreMesh(
    core_axis_name="c", subcore_axis_name="s", num_cores=NC, num_subcores=NS)
_HIGH = lax.Precision.HIGHEST


def _dot(a, b):
    return lax.dot_general(a, b, (((1,), (0,)), ((), ())), precision=_HIGH)


CD = 128
TOTD = 80
EPTD = TOTD * CD
EPADD = NW * EPTD
TCHD = NW * TOTD
DSK = 8
DSUP = TOTD // DSK

NDEG = 10240
DRPT = NDEG // NS


def _deg_body(col2_hbm, w_hbm, out_hbm, col_s, w_s, zb, deg_sh, sem):
    cid = lax.axis_index("c")
    sid = lax.axis_index("s")
    wid = cid * NS + sid
    zv = jnp.zeros((LANES,), F32)

    def zloop(j, _):
        zb[pl.ds(j * LANES, LANES)] = zv
        return 0
    lax.fori_loop(0, DRPT // LANES, zloop, 0)
    pltpu.sync_copy(zb, deg_sh.at[pl.ds(sid * DRPT, DRPT)])
    plsc.subcore_barrier()

    def body(s, _):
        ebase = wid * EPTD + s * DSK * CD
        cbase = wid * TOTD + s * DSK
        pltpu.sync_copy(col2_hbm.at[pl.ds(cbase, DSK)], col_s)
        pltpu.sync_copy(w_hbm.at[pl.ds(ebase, DSK * CD)], w_s)
        descs = []
        for j in range(DSK):
            descs.append(pltpu.async_copy(
                w_s.at[pl.ds(j * CD, CD)], deg_sh.at[col_s.at[j]], sem,
                add=True))
        for d in descs:
            d.wait()
        return 0
    lax.fori_loop(0, DSUP, body, 0)
    plsc.subcore_barrier()
    pltpu.sync_copy(deg_sh.at[pl.ds(sid * DRPT, DRPT)], zb)
    pltpu.sync_copy(zb, out_hbm.at[cid, pl.ds(sid * DRPT, DRPT)])


_deg_call = pl.kernel(
    _deg_body,
    out_type=jax.ShapeDtypeStruct((NC, NDEG), F32),
    mesh=_mesh,
    scratch_types=[
        pltpu.VMEM((DSK, CD), jnp.int32),
        pltpu.VMEM((DSK * CD,), F32),
        pltpu.VMEM((DRPT,), F32),
        pltpu.VMEM_SHARED((NDEG,), F32),
        pltpu.SemaphoreType.DMA,
    ],
)

R3 = 3
SOUT = TOT // R3


def _mp_body(y_hbm, row_hbm, col_hbm, w_hbm, out_hbm,
             r0, r1, r2, c0_, c1_, c2_, w0, w1, w2,
             g0, g1, g2, acc_sh,
             rs0, rs1, rs2, cs0, cs1, cs2, ws0, ws1, ws2,
             gs0, gs1, gs2, ss0, ss1, ss2):
    cid = lax.axis_index("c")
    sid = lax.axis_index("s")
    wid = cid * NS + sid
    zv = jnp.zeros((LANES,), F32)
    rr = (r0, r1, r2)
    cr = (c0_, c1_, c2_)
    wr = (w0, w1, w2)
    gb = (g0, g1, g2)
    rsem = (rs0, rs1, rs2)
    csem = (cs0, cs1, cs2)
    wsem = (ws0, ws1, ws2)
    gsem = (gs0, gs1, gs2)
    ssem = (ss0, ss1, ss2)
    eb0 = wid * EPT

    def zg(e, _):
        for k in range(KH):
            g0[e, pl.ds(k * LANES, LANES)] = zv
        return 0
    lax.fori_loop(0, C, zg, 0)

    def za(j, _):
        pltpu.sync_copy(g0.at[pl.ds(0, WBC)],
                        acc_sh.at[pl.ds(sid * RPT + j * WBC, WBC)])
        return 0
    lax.fori_loop(0, NWB, za, 0)
    plsc.subcore_barrier()

    def idx_load(c, slot):
        base = pl.multiple_of(eb0 + c * C, 8)
        pltpu.async_copy(row_hbm.at[pl.ds(base, C)], rr[slot], rsem[slot])
        pltpu.async_copy(col_hbm.at[pl.ds(base, C)], cr[slot], csem[slot])
        pltpu.async_copy(w_hbm.at[pl.ds(base, C)], wr[slot].at[pl.ds(0, C)],
                         wsem[slot])

    def idx_wait(slot):
        pltpu.make_async_copy(row_hbm.at[pl.ds(0, C)], rr[slot],
                              rsem[slot]).wait()
        pltpu.make_async_copy(col_hbm.at[pl.ds(0, C)], cr[slot],
                              csem[slot]).wait()
        pltpu.make_async_copy(w_hbm.at[pl.ds(0, C)], wr[slot].at[pl.ds(0, C)],
                              wsem[slot]).wait()

    def gath(slot):
        pltpu.async_copy(y_hbm.at[rr[slot]], gb[slot], gsem[slot])

    def gath_wait(slot):
        pltpu.make_async_copy(y_hbm.at[rr[slot]], gb[slot],
                              gsem[slot]).wait()

    def scat(slot):
        pltpu.async_copy(gb[slot], acc_sh.at[cr[slot]], ssem[slot],
                         add=True)

    def scat_wait(slot):
        pltpu.make_async_copy(gb[slot], acc_sh.at[cr[slot]],
                              ssem[slot]).wait()

    idx_load(0, 0)
    idx_load(1, 1)
    idx_wait(0)
    gath(0)

    def body(s, _):
        for j3 in range(R3):
            c0 = s * R3 + j3
            b = j3
            b1 = (j3 + 1) % R3
            b2 = (j3 + 2) % R3

            def do_a():
                idx_wait(b1)
                gath(b1)
            if j3 == R3 - 1:
                @pl.when(s < SOUT - 1)
                def _():
                    do_a()
            else:
                do_a()

            def do_b():
                scat_wait(b2)
                idx_load(c0 + 2, b2)
            if j3 == 0:
                @pl.when(s >= 1)
                def _():
                    scat_wait(b2)
                idx_load(c0 + 2, b2)
            else:
                @pl.when(s < SOUT - 1)
                def _():
                    do_b()

            gath_wait(b)
            buf = gb[b]
            wref = wr[b]

            def scale(g, _):
                wv16 = wref[pl.ds(g * LANES, LANES)]
                for e2_ in range(LANES):
                    wbc = jnp.full((LANES,), wv16[e2_], F32)
                    e = g * LANES + e2_
                    for k in range(KH):
                        sl = pl.ds(k * LANES, LANES)
                        buf[e, sl] = buf[e, sl] * wbc
                return 0
            lax.fori_loop(0, C // LANES, scale, 0)
            wv16t = wref[pl.ds((C // LANES) * LANES, LANES)]
            for e2_ in range(C - (C // LANES) * LANES):
                wbc = jnp.full((LANES,), wv16t[e2_], F32)
                e = (C // LANES) * LANES + e2_
                for k in range(KH):
                    sl = pl.ds(k * LANES, LANES)
                    buf[e, sl] = buf[e, sl] * wbc
            scat(b)
        return 0
    lax.fori_loop(0, SOUT, body, 0)
    for j in range(R3):
        scat_wait(j)
    plsc.subcore_barrier()

    def wb(j, _):
        pltpu.sync_copy(acc_sh.at[pl.ds(sid * RPT + j * WBC, WBC)],
                        g0.at[pl.ds(0, WBC)])
        pltpu.sync_copy(g0.at[pl.ds(0, WBC)],
                        out_hbm.at[cid, pl.ds(sid * RPT + j * WBC, WBC)])
        return 0
    lax.fori_loop(0, NWB, wb, 0)


_mp_call = pl.kernel(
    _mp_body,
    out_type=jax.ShapeDtypeStruct((NC, NACC, H), F32),
    mesh=_mesh,
    scratch_types=[
        pltpu.VMEM((C,), jnp.int32),
        pltpu.VMEM((C,), jnp.int32),
        pltpu.VMEM((C,), jnp.int32),
        pltpu.VMEM((C,), jnp.int32),
        pltpu.VMEM((C,), jnp.int32),
        pltpu.VMEM((C,), jnp.int32),
        pltpu.VMEM((128,), F32),
        pltpu.VMEM((128,), F32),
        pltpu.VMEM((128,), F32),
        pltpu.VMEM((C, H), F32),
        pltpu.VMEM((C, H), F32),
        pltpu.VMEM((C, H), F32),
        pltpu.VMEM_SHARED((NACC, H), F32),
    ] + [pltpu.SemaphoreType.DMA] * 15,
)


def _xt_body(x_ref, w1_ref, xt_ref):
    xt_ref[...] = _dot(x_ref[...], w1_ref[...])


_xt_call = pl.pallas_call(
    _xt_body,
    grid=(NBLK,),
    in_specs=[
        pl.BlockSpec((BLK, 4), lambda i: (i, 0)),
        pl.BlockSpec((4, H), lambda i: (0, 0)),
    ],
    out_specs=pl.BlockSpec((BLK, H), lambda i: (i, 0)),
    out_shape=jax.ShapeDtypeStruct((N, H), F32),
)


def _pre_body(xt_ref, degp_ref, y_ref, dis_ref):
    dp = degp_ref[...]
    deg = dp[0] + dp[1] + 1.0
    dis = lax.rsqrt(deg)
    y_ref[...] = xt_ref[...] * dis
    dis_ref[...] = dis


_pre_call = pl.pallas_call(
    _pre_body,
    grid=(NBLK,),
    in_specs=[
        pl.BlockSpec((BLK, H), lambda i: (i, 0)),
        pl.BlockSpec((NC, BLK, 1), lambda i: (0, i, 0)),
    ],
    out_specs=[
        pl.BlockSpec((BLK, H), lambda i: (i, 0)),
        pl.BlockSpec((BLK, 1), lambda i: (i, 0)),
    ],
    out_shape=[
        jax.ShapeDtypeStruct((N, H), F32),
        jax.ShapeDtypeStruct((N, 1), F32),
    ],
)


def _stage_body(acc_ref, y_ref, dis_ref, b_ref, w_ref, yn_ref):
    a = acc_ref[0] + acc_ref[1] + y_ref[...]
    h = a * dis_ref[...] + b_ref[...]
    h = jnp.where(h > 0, h, 0.01 * h)
    yn_ref[...] = _dot(h, w_ref[...]) * dis_ref[...]


_stage_call = pl.pallas_call(
    _stage_body,
    grid=(NBLK,),
    in_specs=[
        pl.BlockSpec((NC, BLK, H), lambda i: (0, i, 0)),
        pl.BlockSpec((BLK, H), lambda i: (i, 0)),
        pl.BlockSpec((BLK, 1), lambda i: (i, 0)),
        pl.BlockSpec((1, H), lambda i: (0, 0)),
        pl.BlockSpec((H, H), lambda i: (0, 0)),
    ],
    out_specs=pl.BlockSpec((BLK, H), lambda i: (i, 0)),
    out_shape=jax.ShapeDtypeStruct((N, H), F32),
)


def _final_body(acc_ref, y_ref, dis_ref, b_ref, batch_ref,
                l1w_ref, l1b_ref, l2w_ref, l2b_ref, l3w_ref, l3b_ref,
                out_ref, pool_ref):
    i = pl.program_id(0)
    a = acc_ref[0] + acc_ref[1] + y_ref[...]
    h = a * dis_ref[...] + b_ref[...]
    h = jnp.where(h > 0, h, 0.01 * h)
    gi = lax.broadcasted_iota(jnp.int32, (BLK, G), 1)
    oh = jnp.where(batch_ref[...] == gi, 1.0, 0.0)
    contrib = lax.dot_general(oh, h, (((0,), (0,)), ((), ())),
                              precision=_HIGH)

    @pl.when(i == 0)
    def _():
        pool_ref[...] = contrib

    @pl.when(i > 0)
    def _():
        pool_ref[...] += contrib

    @pl.when(i == NBLK - 1)
    def _():
        z = jnp.maximum(_dot(pool_ref[...], l1w_ref[...]) + l1b_ref[...], 0.0)
        z = jnp.maximum(_dot(z, l2w_ref[...]) + l2b_ref[...], 0.0)
        out_ref[...] = _dot(z, l3w_ref[...]) + l3b_ref[...]


_final_call = pl.pallas_call(
    _final_body,
    grid=(NBLK,),
    in_specs=[
        pl.BlockSpec((NC, BLK, H), lambda i: (0, i, 0)),
        pl.BlockSpec((BLK, H), lambda i: (i, 0)),
        pl.BlockSpec((BLK, 1), lambda i: (i, 0)),
        pl.BlockSpec((1, H), lambda i: (0, 0)),
        pl.BlockSpec((BLK, 1), lambda i: (i, 0)),
        pl.BlockSpec((H, 256), lambda i: (0, 0)),
        pl.BlockSpec((1, 256), lambda i: (0, 0)),
        pl.BlockSpec((256, H), lambda i: (0, 0)),
        pl.BlockSpec((1, H), lambda i: (0, 0)),
        pl.BlockSpec((H, 2), lambda i: (0, 0)),
        pl.BlockSpec((1, 2), lambda i: (0, 0)),
    ],
    out_specs=pl.BlockSpec((G, 2), lambda i: (0, 0)),
    out_shape=jax.ShapeDtypeStruct((G, 2), F32),
    scratch_shapes=[pltpu.VMEM((G, H), F32)],
)


def kernel(x, edge_index, edge_attr, batch, W1, b1, W2, b2,
           L1W, L1b, L2W, L2b, L3W, L3b):
    row = edge_index[0].astype(jnp.int32)
    col = edge_index[1].astype(jnp.int32)
    npe = EPAD - E
    fill = (jnp.arange(npe, dtype=jnp.int32) * 37) % N
    rowp = jnp.concatenate([row, fill])
    colp = jnp.concatenate([col, fill])
    wp = jnp.concatenate([edge_attr.astype(F32), jnp.zeros((npe,), F32)])
    nped = EPADD - E
    filld = (jnp.arange(nped, dtype=jnp.int32) * 37) % N
    col2d = jnp.concatenate([col, filld]).reshape(TCHD, CD)
    wpd = jnp.concatenate([edge_attr.astype(F32), jnp.zeros((nped,), F32)])
    batchp = batch.astype(jnp.int32)[:, None]

    xt1 = _xt_call(x.astype(F32), W1)
    degp = _deg_call(col2d, wpd)[:, :N, None]
    y1, dis = _pre_call(xt1, degp)
    acc1 = _mp_call(y1, rowp, colp, wp)
    y2 = _stage_call(acc1, y1, dis, b1[None, :], W2)
    acc2 = _mp_call(y2, rowp, colp, wp)
    y3 = _stage_call(acc2, y2, dis, b2[None, :], W2)
    acc3 = _mp_call(y3, rowp, colp, wp)
    z = _final_call(acc3, y3, dis, b2[None, :], batchp,
                    L1W, L1b[None, :], L2W, L2b[None, :], L3W, L3b[None, :])
    return z.reshape(-1)


# speedup vs baseline: 1.0039x; 1.0039x over previous
import jax
import jax.numpy as jnp
from jax import lax
from jax.experimental import pallas as pl
from jax.experimental.pallas import tpu as pltpu
from jax.experimental.pallas import tpu_sc as plsc

N, E, H, G = 10000, 320000, 128, 64
NC, NS, LANES = 2, 16, 16
NW = NC * NS
C = 112
TOT = 90
EPT = TOT * C
EPAD = NW * EPT
TCH = NW * TOT
NACC = 10240
RPT = NACC // NS
WBC = 80
NWB = RPT // WBC
KH = H // LANES
BLK = 2000
NBLK = N // BLK
F32 = jnp.float32

_mesh = plsc.VectorSubcoreMesh(
    core_axis_name="c", subcore_axis_name="s", num_cores=NC, num_subcores=NS)
_HIGH = lax.Precision.HIGHEST


def _dot(a, b):
    return lax.dot_general(a, b, (((1,), (0,)), ((), ())), precision=_HIGH)


CD = 128
TOTD = 80
EPTD = TOTD * CD
EPADD = NW * EPTD
TCHD = NW * TOTD
DSK = 8
DSUP = TOTD // DSK

NDEG = 10240
DRPT = NDEG // NS


def _deg_body(col2_hbm, w_hbm, out_hbm, col_s, w_s, zb, deg_sh, sem):
    cid = lax.axis_index("c")
    sid = lax.axis_index("s")
    wid = cid * NS + sid
    zv = jnp.zeros((LANES,), F32)

    def zloop(j, _):
        zb[pl.ds(j * LANES, LANES)] = zv
        return 0
    lax.fori_loop(0, DRPT // LANES, zloop, 0)
    pltpu.sync_copy(zb, deg_sh.at[pl.ds(sid * DRPT, DRPT)])
    plsc.subcore_barrier()

    def body(s, _):
        ebase = wid * EPTD + s * DSK * CD
        cbase = wid * TOTD + s * DSK
        pltpu.sync_copy(col2_hbm.at[pl.ds(cbase, DSK)], col_s)
        pltpu.sync_copy(w_hbm.at[pl.ds(ebase, DSK * CD)], w_s)
        descs = []
        for j in range(DSK):
            descs.append(pltpu.async_copy(
                w_s.at[pl.ds(j * CD, CD)], deg_sh.at[col_s.at[j]], sem,
                add=True))
        for d in descs:
            d.wait()
        return 0
    lax.fori_loop(0, DSUP, body, 0)
    plsc.subcore_barrier()
    pltpu.sync_copy(deg_sh.at[pl.ds(sid * DRPT, DRPT)], zb)
    pltpu.sync_copy(zb, out_hbm.at[cid, pl.ds(sid * DRPT, DRPT)])


_deg_call = pl.kernel(
    _deg_body,
    out_type=jax.ShapeDtypeStruct((NC, NDEG), F32),
    mesh=_mesh,
    scratch_types=[
        pltpu.VMEM((DSK, CD), jnp.int32),
        pltpu.VMEM((DSK * CD,), F32),
        pltpu.VMEM((DRPT,), F32),
        pltpu.VMEM_SHARED((NDEG,), F32),
        pltpu.SemaphoreType.DMA,
    ],
)

R3 = 3
SOUT = TOT // R3


def _mp_body(y_hbm, row_hbm, col_hbm, w_hbm, out_hbm,
             r0, r1, r2, c0_, c1_, c2_, w0, w1, w2,
             g0, g1, g2, acc_sh,
             rs0, rs1, rs2, cs0, cs1, cs2, ws0, ws1, ws2,
             gs0, gs1, gs2, ss0, ss1, ss2):
    cid = lax.axis_index("c")
    sid = lax.axis_index("s")
    wid = cid * NS + sid
    zv = jnp.zeros((LANES,), F32)
    rr = (r0, r1, r2)
    cr = (c0_, c1_, c2_)
    wr = (w0, w1, w2)
    gb = (g0, g1, g2)
    rsem = (rs0, rs1, rs2)
    csem = (cs0, cs1, cs2)
    wsem = (ws0, ws1, ws2)
    gsem = (gs0, gs1, gs2)
    ssem = (ss0, ss1, ss2)
    eb0 = wid * EPT

    def zg(e, _):
        for k in range(KH):
            g0[e, pl.ds(k * LANES, LANES)] = zv
        return 0
    lax.fori_loop(0, C, zg, 0)

    def za(j, _):
        pltpu.sync_copy(g0.at[pl.ds(0, WBC)],
                        acc_sh.at[pl.ds(sid * RPT + j * WBC, WBC)])
        return 0
    lax.fori_loop(0, NWB, za, 0)
    plsc.subcore_barrier()

    def idx_load(c, slot):
        base = pl.multiple_of(eb0 + c * C, 8)
        pltpu.async_copy(row_hbm.at[pl.ds(base, C)], rr[slot], rsem[slot])
        pltpu.async_copy(col_hbm.at[pl.ds(base, C)], cr[slot], csem[slot])
        pltpu.async_copy(w_hbm.at[pl.ds(base, C)], wr[slot], wsem[slot])

    def idx_wait(slot):
        pltpu.make_async_copy(row_hbm.at[pl.ds(0, C)], rr[slot],
                              rsem[slot]).wait()
        pltpu.make_async_copy(col_hbm.at[pl.ds(0, C)], cr[slot],
                              csem[slot]).wait()
        pltpu.make_async_copy(w_hbm.at[pl.ds(0, C)], wr[slot],
                              wsem[slot]).wait()

    def gath(slot):
        pltpu.async_copy(y_hbm.at[rr[slot]], gb[slot], gsem[slot])

    def gath_wait(slot):
        pltpu.make_async_copy(y_hbm.at[rr[slot]], gb[slot],
                              gsem[slot]).wait()

    def scat(slot):
        pltpu.async_copy(gb[slot], acc_sh.at[cr[slot]], ssem[slot],
                         add=True)

    def scat_wait(slot):
        pltpu.make_async_copy(gb[slot], acc_sh.at[cr[slot]],
                              ssem[slot]).wait()

    idx_load(0, 0)
    idx_load(1, 1)
    idx_wait(0)
    gath(0)

    def body(s, _):
        for j3 in range(R3):
            c0 = s * R3 + j3
            b = j3
            b1 = (j3 + 1) % R3
            b2 = (j3 + 2) % R3

            def do_a():
                idx_wait(b1)
                gath(b1)
            if j3 == R3 - 1:
                @pl.when(s < SOUT - 1)
                def _():
                    do_a()
            else:
                do_a()

            def do_b():
                scat_wait(b2)
                idx_load(c0 + 2, b2)
            if j3 == 0:
                @pl.when(s >= 1)
                def _():
                    scat_wait(b2)
                idx_load(c0 + 2, b2)
            else:
                @pl.when(s < SOUT - 1)
                def _():
                    do_b()

            gath_wait(b)
            buf = gb[b]
            wref = wr[b]

            def scale(g, _):
                wv16 = wref[pl.ds(g * LANES, LANES)]
                for e2_ in range(LANES):
                    wbc = jnp.full((LANES,), wv16[e2_], F32)
                    e = g * LANES + e2_
                    for k in range(KH):
                        sl = pl.ds(k * LANES, LANES)
                        buf[e, sl] = buf[e, sl] * wbc
                return 0
            lax.fori_loop(0, C // LANES, scale, 0)
            scat(b)
        return 0
    lax.fori_loop(0, SOUT, body, 0)
    for j in range(R3):
        scat_wait(j)
    plsc.subcore_barrier()

    def wb(j, _):
        pltpu.sync_copy(acc_sh.at[pl.ds(sid * RPT + j * WBC, WBC)],
                        g0.at[pl.ds(0, WBC)])
        pltpu.sync_copy(g0.at[pl.ds(0, WBC)],
                        out_hbm.at[cid, pl.ds(sid * RPT + j * WBC, WBC)])
        return 0
    lax.fori_loop(0, NWB, wb, 0)


_mp_call = pl.kernel(
    _mp_body,
    out_type=jax.ShapeDtypeStruct((NC, NACC, H), F32),
    mesh=_mesh,
    scratch_types=[
        pltpu.VMEM((C,), jnp.int32),
        pltpu.VMEM((C,), jnp.int32),
        pltpu.VMEM((C,), jnp.int32),
        pltpu.VMEM((C,), jnp.int32),
        pltpu.VMEM((C,), jnp.int32),
        pltpu.VMEM((C,), jnp.int32),
        pltpu.VMEM((C,), F32),
        pltpu.VMEM((C,), F32),
        pltpu.VMEM((C,), F32),
        pltpu.VMEM((C, H), F32),
        pltpu.VMEM((C, H), F32),
        pltpu.VMEM((C, H), F32),
        pltpu.VMEM_SHARED((NACC, H), F32),
    ] + [pltpu.SemaphoreType.DMA] * 15,
)


def _xt_body(x_ref, w1_ref, xt_ref):
    xt_ref[...] = _dot(x_ref[...], w1_ref[...])


_xt_call = pl.pallas_call(
    _xt_body,
    grid=(NBLK,),
    in_specs=[
        pl.BlockSpec((BLK, 4), lambda i: (i, 0)),
        pl.BlockSpec((4, H), lambda i: (0, 0)),
    ],
    out_specs=pl.BlockSpec((BLK, H), lambda i: (i, 0)),
    out_shape=jax.ShapeDtypeStruct((N, H), F32),
)


def _pre_body(xt_ref, degp_ref, y_ref, dis_ref):
    dp = degp_ref[...]
    deg = dp[0] + dp[1] + 1.0
    dis = lax.rsqrt(deg)
    y_ref[...] = xt_ref[...] * dis
    dis_ref[...] = dis


_pre_call = pl.pallas_call(
    _pre_body,
    grid=(NBLK,),
    in_specs=[
        pl.BlockSpec((BLK, H), lambda i: (i, 0)),
        pl.BlockSpec((NC, BLK, 1), lambda i: (0, i, 0)),
    ],
    out_specs=[
        pl.BlockSpec((BLK, H), lambda i: (i, 0)),
        pl.BlockSpec((BLK, 1), lambda i: (i, 0)),
    ],
    out_shape=[
        jax.ShapeDtypeStruct((N, H), F32),
        jax.ShapeDtypeStruct((N, 1), F32),
    ],
)


def _stage_body(acc_ref, y_ref, dis_ref, b_ref, w_ref, yn_ref):
    a = acc_ref[0] + acc_ref[1] + y_ref[...]
    h = a * dis_ref[...] + b_ref[...]
    h = jnp.where(h > 0, h, 0.01 * h)
    yn_ref[...] = _dot(h, w_ref[...]) * dis_ref[...]


_stage_call = pl.pallas_call(
    _stage_body,
    grid=(NBLK,),
    in_specs=[
        pl.BlockSpec((NC, BLK, H), lambda i: (0, i, 0)),
        pl.BlockSpec((BLK, H), lambda i: (i, 0)),
        pl.BlockSpec((BLK, 1), lambda i: (i, 0)),
        pl.BlockSpec((1, H), lambda i: (0, 0)),
        pl.BlockSpec((H, H), lambda i: (0, 0)),
    ],
    out_specs=pl.BlockSpec((BLK, H), lambda i: (i, 0)),
    out_shape=jax.ShapeDtypeStruct((N, H), F32),
)


def _final_body(acc_ref, y_ref, dis_ref, b_ref, batch_ref,
                l1w_ref, l1b_ref, l2w_ref, l2b_ref, l3w_ref, l3b_ref,
                out_ref, pool_ref):
    i = pl.program_id(0)
    a = acc_ref[0] + acc_ref[1] + y_ref[...]
    h = a * dis_ref[...] + b_ref[...]
    h = jnp.where(h > 0, h, 0.01 * h)
    gi = lax.broadcasted_iota(jnp.int32, (BLK, G), 1)
    oh = jnp.where(batch_ref[...] == gi, 1.0, 0.0)
    contrib = lax.dot_general(oh, h, (((0,), (0,)), ((), ())),
                              precision=_HIGH)

    @pl.when(i == 0)
    def _():
        pool_ref[...] = contrib

    @pl.when(i > 0)
    def _():
        pool_ref[...] += contrib

    @pl.when(i == NBLK - 1)
    def _():
        z = jnp.maximum(_dot(pool_ref[...], l1w_ref[...]) + l1b_ref[...], 0.0)
        z = jnp.maximum(_dot(z, l2w_ref[...]) + l2b_ref[...], 0.0)
        out_ref[...] = _dot(z, l3w_ref[...]) + l3b_ref[...]


_final_call = pl.pallas_call(
    _final_body,
    grid=(NBLK,),
    in_specs=[
        pl.BlockSpec((NC, BLK, H), lambda i: (0, i, 0)),
        pl.BlockSpec((BLK, H), lambda i: (i, 0)),
        pl.BlockSpec((BLK, 1), lambda i: (i, 0)),
        pl.BlockSpec((1, H), lambda i: (0, 0)),
        pl.BlockSpec((BLK, 1), lambda i: (i, 0)),
        pl.BlockSpec((H, 256), lambda i: (0, 0)),
        pl.BlockSpec((1, 256), lambda i: (0, 0)),
        pl.BlockSpec((256, H), lambda i: (0, 0)),
        pl.BlockSpec((1, H), lambda i: (0, 0)),
        pl.BlockSpec((H, 2), lambda i: (0, 0)),
        pl.BlockSpec((1, 2), lambda i: (0, 0)),
    ],
    out_specs=pl.BlockSpec((G, 2), lambda i: (0, 0)),
    out_shape=jax.ShapeDtypeStruct((G, 2), F32),
    scratch_shapes=[pltpu.VMEM((G, H), F32)],
)


def kernel(x, edge_index, edge_attr, batch, W1, b1, W2, b2,
           L1W, L1b, L2W, L2b, L3W, L3b):
    row = edge_index[0].astype(jnp.int32)
    col = edge_index[1].astype(jnp.int32)
    npe = EPAD - E
    fill = (jnp.arange(npe, dtype=jnp.int32) * 37) % N
    rowp = jnp.concatenate([row, fill])
    colp = jnp.concatenate([col, fill])
    wp = jnp.concatenate([edge_attr.astype(F32), jnp.zeros((npe,), F32)])
    nped = EPADD - E
    filld = (jnp.arange(nped, dtype=jnp.int32) * 37) % N
    col2d = jnp.concatenate([col, filld]).reshape(TCHD, CD)
    wpd = jnp.concatenate([edge_attr.astype(F32), jnp.zeros((nped,), F32)])
    batchp = batch.astype(jnp.int32)[:, None]

    xt1 = _xt_call(x.astype(F32), W1)
    degp = _deg_call(col2d, wpd)[:, :N, None]
    y1, dis = _pre_call(xt1, degp)
    acc1 = _mp_call(y1, rowp, colp, wp)
    y2 = _stage_call(acc1, y1, dis, b1[None, :], W2)
    acc2 = _mp_call(y2, rowp, colp, wp)
    y3 = _stage_call(acc2, y2, dis, b2[None, :], W2)
    acc3 = _mp_call(y3, rowp, colp, wp)
    z = _final_call(acc3, y3, dis, b2[None, :], batchp,
                    L1W, L1b[None, :], L2W, L2b[None, :], L3W, L3b[None, :])
    return z.reshape(-1)


# speedup vs baseline: 1.0071x; 1.0032x over previous
import jax
import jax.numpy as jnp
from jax import lax
from jax.experimental import pallas as pl
from jax.experimental.pallas import tpu as pltpu
from jax.experimental.pallas import tpu_sc as plsc

N, E, H, G = 10000, 320000, 128, 64
NC, NS, LANES = 2, 16, 16
NW = NC * NS
C = 112
TOT = 90
EPT = TOT * C
EPAD = NW * EPT
TCH = NW * TOT
NACC = 10240
RPT = NACC // NS
WBC = 80
NWB = RPT // WBC
KH = H // LANES
BLK = 2000
NBLK = N // BLK
F32 = jnp.float32

_mesh = plsc.VectorSubcoreMesh(
    core_axis_name="c", subcore_axis_name="s", num_cores=NC, num_subcores=NS)
_HIGH = lax.Precision.HIGHEST


def _dot(a, b):
    return lax.dot_general(a, b, (((1,), (0,)), ((), ())))


CD = 128
TOTD = 80
EPTD = TOTD * CD
EPADD = NW * EPTD
TCHD = NW * TOTD
DSK = 8
DSUP = TOTD // DSK

NDEG = 10240
DRPT = NDEG // NS


def _deg_body(col2_hbm, w_hbm, out_hbm, col_s, w_s, zb, deg_sh, sem):
    cid = lax.axis_index("c")
    sid = lax.axis_index("s")
    wid = cid * NS + sid
    zv = jnp.zeros((LANES,), F32)

    def zloop(j, _):
        zb[pl.ds(j * LANES, LANES)] = zv
        return 0
    lax.fori_loop(0, DRPT // LANES, zloop, 0)
    pltpu.sync_copy(zb, deg_sh.at[pl.ds(sid * DRPT, DRPT)])
    plsc.subcore_barrier()

    def body(s, _):
        ebase = wid * EPTD + s * DSK * CD
        cbase = wid * TOTD + s * DSK
        pltpu.sync_copy(col2_hbm.at[pl.ds(cbase, DSK)], col_s)
        pltpu.sync_copy(w_hbm.at[pl.ds(ebase, DSK * CD)], w_s)
        descs = []
        for j in range(DSK):
            descs.append(pltpu.async_copy(
                w_s.at[pl.ds(j * CD, CD)], deg_sh.at[col_s.at[j]], sem,
                add=True))
        for d in descs:
            d.wait()
        return 0
    lax.fori_loop(0, DSUP, body, 0)
    plsc.subcore_barrier()
    pltpu.sync_copy(deg_sh.at[pl.ds(sid * DRPT, DRPT)], zb)
    pltpu.sync_copy(zb, out_hbm.at[cid, pl.ds(sid * DRPT, DRPT)])


_deg_call = pl.kernel(
    _deg_body,
    out_type=jax.ShapeDtypeStruct((NC, NDEG), F32),
    mesh=_mesh,
    scratch_types=[
        pltpu.VMEM((DSK, CD), jnp.int32),
        pltpu.VMEM((DSK * CD,), F32),
        pltpu.VMEM((DRPT,), F32),
        pltpu.VMEM_SHARED((NDEG,), F32),
        pltpu.SemaphoreType.DMA,
    ],
)

R3 = 3
SOUT = TOT // R3


def _mp_body(y_hbm, row_hbm, col_hbm, w_hbm, out_hbm,
             r0, r1, r2, c0_, c1_, c2_, w0, w1, w2,
             g0, g1, g2, acc_sh,
             rs0, rs1, rs2, cs0, cs1, cs2, ws0, ws1, ws2,
             gs0, gs1, gs2, ss0, ss1, ss2):
    cid = lax.axis_index("c")
    sid = lax.axis_index("s")
    wid = cid * NS + sid
    zv = jnp.zeros((LANES,), F32)
    rr = (r0, r1, r2)
    cr = (c0_, c1_, c2_)
    wr = (w0, w1, w2)
    gb = (g0, g1, g2)
    rsem = (rs0, rs1, rs2)
    csem = (cs0, cs1, cs2)
    wsem = (ws0, ws1, ws2)
    gsem = (gs0, gs1, gs2)
    ssem = (ss0, ss1, ss2)
    eb0 = wid * EPT

    def zg(e, _):
        for k in range(KH):
            g0[e, pl.ds(k * LANES, LANES)] = zv
        return 0
    lax.fori_loop(0, C, zg, 0)

    def za(j, _):
        pltpu.sync_copy(g0.at[pl.ds(0, WBC)],
                        acc_sh.at[pl.ds(sid * RPT + j * WBC, WBC)])
        return 0
    lax.fori_loop(0, NWB, za, 0)
    plsc.subcore_barrier()

    def idx_load(c, slot):
        base = pl.multiple_of(eb0 + c * C, 8)
        pltpu.async_copy(row_hbm.at[pl.ds(base, C)], rr[slot], rsem[slot])
        pltpu.async_copy(col_hbm.at[pl.ds(base, C)], cr[slot], csem[slot])
        pltpu.async_copy(w_hbm.at[pl.ds(base, C)], wr[slot], wsem[slot])

    def idx_wait(slot):
        pltpu.make_async_copy(row_hbm.at[pl.ds(0, C)], rr[slot],
                              rsem[slot]).wait()
        pltpu.make_async_copy(col_hbm.at[pl.ds(0, C)], cr[slot],
                              csem[slot]).wait()
        pltpu.make_async_copy(w_hbm.at[pl.ds(0, C)], wr[slot],
                              wsem[slot]).wait()

    def gath(slot):
        pltpu.async_copy(y_hbm.at[rr[slot]], gb[slot], gsem[slot])

    def gath_wait(slot):
        pltpu.make_async_copy(y_hbm.at[rr[slot]], gb[slot],
                              gsem[slot]).wait()

    def scat(slot):
        pltpu.async_copy(gb[slot], acc_sh.at[cr[slot]], ssem[slot],
                         add=True)

    def scat_wait(slot):
        pltpu.make_async_copy(gb[slot], acc_sh.at[cr[slot]],
                              ssem[slot]).wait()

    idx_load(0, 0)
    idx_load(1, 1)
    idx_wait(0)
    gath(0)

    def body(s, _):
        for j3 in range(R3):
            c0 = s * R3 + j3
            b = j3
            b1 = (j3 + 1) % R3
            b2 = (j3 + 2) % R3

            def do_a():
                idx_wait(b1)
                gath(b1)
            if j3 == R3 - 1:
                @pl.when(s < SOUT - 1)
                def _():
                    do_a()
            else:
                do_a()

            def do_b():
                scat_wait(b2)
                idx_load(c0 + 2, b2)
            if j3 == 0:
                @pl.when(s >= 1)
                def _():
                    scat_wait(b2)
                idx_load(c0 + 2, b2)
            else:
                @pl.when(s < SOUT - 1)
                def _():
                    do_b()

            gath_wait(b)
            buf = gb[b]
            wref = wr[b]

            def scale(g, _):
                wv16 = wref[pl.ds(g * LANES, LANES)]
                for e2_ in range(LANES):
                    wbc = jnp.full((LANES,), wv16[e2_], F32)
                    e = g * LANES + e2_
                    for k in range(KH):
                        sl = pl.ds(k * LANES, LANES)
                        buf[e, sl] = buf[e, sl] * wbc
                return 0
            lax.fori_loop(0, C // LANES, scale, 0)
            scat(b)
        return 0
    lax.fori_loop(0, SOUT, body, 0)
    for j in range(R3):
        scat_wait(j)
    plsc.subcore_barrier()

    def wb(j, _):
        pltpu.sync_copy(acc_sh.at[pl.ds(sid * RPT + j * WBC, WBC)],
                        g0.at[pl.ds(0, WBC)])
        pltpu.sync_copy(g0.at[pl.ds(0, WBC)],
                        out_hbm.at[cid, pl.ds(sid * RPT + j * WBC, WBC)])
        return 0
    lax.fori_loop(0, NWB, wb, 0)


_mp_call = pl.kernel(
    _mp_body,
    out_type=jax.ShapeDtypeStruct((NC, NACC, H), F32),
    mesh=_mesh,
    scratch_types=[
        pltpu.VMEM((C,), jnp.int32),
        pltpu.VMEM((C,), jnp.int32),
        pltpu.VMEM((C,), jnp.int32),
        pltpu.VMEM((C,), jnp.int32),
        pltpu.VMEM((C,), jnp.int32),
        pltpu.VMEM((C,), jnp.int32),
        pltpu.VMEM((C,), F32),
        pltpu.VMEM((C,), F32),
        pltpu.VMEM((C,), F32),
        pltpu.VMEM((C, H), F32),
        pltpu.VMEM((C, H), F32),
        pltpu.VMEM((C, H), F32),
        pltpu.VMEM_SHARED((NACC, H), F32),
    ] + [pltpu.SemaphoreType.DMA] * 15,
)


def _xt_body(x_ref, w1_ref, xt_ref):
    xt_ref[...] = _dot(x_ref[...], w1_ref[...])


_xt_call = pl.pallas_call(
    _xt_body,
    grid=(NBLK,),
    in_specs=[
        pl.BlockSpec((BLK, 4), lambda i: (i, 0)),
        pl.BlockSpec((4, H), lambda i: (0, 0)),
    ],
    out_specs=pl.BlockSpec((BLK, H), lambda i: (i, 0)),
    out_shape=jax.ShapeDtypeStruct((N, H), F32),
)


def _pre_body(xt_ref, degp_ref, y_ref, dis_ref):
    dp = degp_ref[...]
    deg = dp[0] + dp[1] + 1.0
    dis = lax.rsqrt(deg)
    y_ref[...] = xt_ref[...] * dis
    dis_ref[...] = dis


_pre_call = pl.pallas_call(
    _pre_body,
    grid=(NBLK,),
    in_specs=[
        pl.BlockSpec((BLK, H), lambda i: (i, 0)),
        pl.BlockSpec((NC, BLK, 1), lambda i: (0, i, 0)),
    ],
    out_specs=[
        pl.BlockSpec((BLK, H), lambda i: (i, 0)),
        pl.BlockSpec((BLK, 1), lambda i: (i, 0)),
    ],
    out_shape=[
        jax.ShapeDtypeStruct((N, H), F32),
        jax.ShapeDtypeStruct((N, 1), F32),
    ],
)


def _stage_body(acc_ref, y_ref, dis_ref, b_ref, w_ref, yn_ref):
    a = acc_ref[0] + acc_ref[1] + y_ref[...]
    h = a * dis_ref[...] + b_ref[...]
    h = jnp.where(h > 0, h, 0.01 * h)
    yn_ref[...] = _dot(h, w_ref[...]) * dis_ref[...]


_stage_call = pl.pallas_call(
    _stage_body,
    grid=(NBLK,),
    in_specs=[
        pl.BlockSpec((NC, BLK, H), lambda i: (0, i, 0)),
        pl.BlockSpec((BLK, H), lambda i: (i, 0)),
        pl.BlockSpec((BLK, 1), lambda i: (i, 0)),
        pl.BlockSpec((1, H), lambda i: (0, 0)),
        pl.BlockSpec((H, H), lambda i: (0, 0)),
    ],
    out_specs=pl.BlockSpec((BLK, H), lambda i: (i, 0)),
    out_shape=jax.ShapeDtypeStruct((N, H), F32),
)


def _final_body(acc_ref, y_ref, dis_ref, b_ref, batch_ref,
                l1w_ref, l1b_ref, l2w_ref, l2b_ref, l3w_ref, l3b_ref,
                out_ref, pool_ref):
    i = pl.program_id(0)
    a = acc_ref[0] + acc_ref[1] + y_ref[...]
    h = a * dis_ref[...] + b_ref[...]
    h = jnp.where(h > 0, h, 0.01 * h)
    gi = lax.broadcasted_iota(jnp.int32, (BLK, G), 1)
    oh = jnp.where(batch_ref[...] == gi, 1.0, 0.0)
    contrib = lax.dot_general(oh, h, (((0,), (0,)), ((), ())),
                              precision=_HIGH)

    @pl.when(i == 0)
    def _():
        pool_ref[...] = contrib

    @pl.when(i > 0)
    def _():
        pool_ref[...] += contrib

    @pl.when(i == NBLK - 1)
    def _():
        z = jnp.maximum(_dot(pool_ref[...], l1w_ref[...]) + l1b_ref[...], 0.0)
        z = jnp.maximum(_dot(z, l2w_ref[...]) + l2b_ref[...], 0.0)
        out_ref[...] = _dot(z, l3w_ref[...]) + l3b_ref[...]


_final_call = pl.pallas_call(
    _final_body,
    grid=(NBLK,),
    in_specs=[
        pl.BlockSpec((NC, BLK, H), lambda i: (0, i, 0)),
        pl.BlockSpec((BLK, H), lambda i: (i, 0)),
        pl.BlockSpec((BLK, 1), lambda i: (i, 0)),
        pl.BlockSpec((1, H), lambda i: (0, 0)),
        pl.BlockSpec((BLK, 1), lambda i: (i, 0)),
        pl.BlockSpec((H, 256), lambda i: (0, 0)),
        pl.BlockSpec((1, 256), lambda i: (0, 0)),
        pl.BlockSpec((256, H), lambda i: (0, 0)),
        pl.BlockSpec((1, H), lambda i: (0, 0)),
        pl.BlockSpec((H, 2), lambda i: (0, 0)),
        pl.BlockSpec((1, 2), lambda i: (0, 0)),
    ],
    out_specs=pl.BlockSpec((G, 2), lambda i: (0, 0)),
    out_shape=jax.ShapeDtypeStruct((G, 2), F32),
    scratch_shapes=[pltpu.VMEM((G, H), F32)],
)


def kernel(x, edge_index, edge_attr, batch, W1, b1, W2, b2,
           L1W, L1b, L2W, L2b, L3W, L3b):
    row = edge_index[0].astype(jnp.int32)
    col = edge_index[1].astype(jnp.int32)
    npe = EPAD - E
    fill = (jnp.arange(npe, dtype=jnp.int32) * 37) % N
    rowp = jnp.concatenate([row, fill])
    colp = jnp.concatenate([col, fill])
    wp = jnp.concatenate([edge_attr.astype(F32), jnp.zeros((npe,), F32)])
    nped = EPADD - E
    filld = (jnp.arange(nped, dtype=jnp.int32) * 37) % N
    col2d = jnp.concatenate([col, filld]).reshape(TCHD, CD)
    wpd = jnp.concatenate([edge_attr.astype(F32), jnp.zeros((nped,), F32)])
    batchp = batch.astype(jnp.int32)[:, None]

    xt1 = _xt_call(x.astype(F32), W1)
    degp = _deg_call(col2d, wpd)[:, :N, None]
    y1, dis = _pre_call(xt1, degp)
    acc1 = _mp_call(y1, rowp, colp, wp)
    y2 = _stage_call(acc1, y1, dis, b1[None, :], W2)
    acc2 = _mp_call(y2, rowp, colp, wp)
    y3 = _stage_call(acc2, y2, dis, b2[None, :], W2)
    acc3 = _mp_call(y3, rowp, colp, wp)
    z = _final_call(acc3, y3, dis, b2[None, :], batchp,
                    L1W, L1b[None, :], L2W, L2b[None, :], L3W, L3b[None, :])
    return z.reshape(-1)
